# baseline (speedup 1.0000x reference)
import functools

import jax
import jax.numpy as jnp
from jax import lax
from jax.experimental import pallas as pl
from jax.experimental.pallas import tpu as pltpu
from jax.experimental.pallas import tpu_sc as plsc

N = 50000
E = 800000
C = 16
K = 27

NC = 2
NS = 16
NW = NC * NS
CHUNK = 128
GP = 4
GROUP_ROWS = GP * CHUNK
CH = 200
NG = CH // GP
E_PAD = NW * CH * CHUNK
N_PAD = N + 48
RPT = N_PAD // NS
ROW_BLK = 2000
DST_PAD = N + 1


def _mm1_body(x_ref, w_ref, src_ref, kid_ref, y_ref, g_ref):
    y_ref[...] = jnp.dot(x_ref[...], w_ref[...],
                         preferred_element_type=jnp.float32)
    g_ref[...] = src_ref[...] * K + kid_ref[...]


def _mm_mid_body(p0_ref, p1_ref, w_ref, y_ref):
    h = jnp.maximum(p0_ref[0] + p1_ref[0], 0.0)
    y_ref[...] = jnp.dot(h, w_ref[...], preferred_element_type=jnp.float32)


def _final_body(p0_ref, p1_ref, o_ref):
    o_ref[...] = jnp.maximum(p0_ref[0] + p1_ref[0], 0.0)


def _mm1(x, wcat, src_pad, kid_pad):
    grid_n = N // ROW_BLK
    irows = E_PAD // 128
    iblk = irows // grid_n
    return pl.pallas_call(
        _mm1_body,
        grid=(grid_n,),
        in_specs=[
            pl.BlockSpec((ROW_BLK, C), lambda i: (i, 0)),
            pl.BlockSpec((C, K * C), lambda i: (0, 0)),
            pl.BlockSpec((iblk, 128), lambda i: (i, 0)),
            pl.BlockSpec((iblk, 128), lambda i: (i, 0)),
        ],
        out_specs=[
            pl.BlockSpec((ROW_BLK, K * C), lambda i: (i, 0)),
            pl.BlockSpec((iblk, 128), lambda i: (i, 0)),
        ],
        out_shape=[
            jax.ShapeDtypeStruct((N, K * C), jnp.float32),
            jax.ShapeDtypeStruct((irows, 128), jnp.int32),
        ],
    )(x, wcat, src_pad.reshape(irows, 128), kid_pad.reshape(irows, 128))


def _mm_mid(partials, wcat):
    return pl.pallas_call(
        _mm_mid_body,
        grid=(N // ROW_BLK,),
        in_specs=[
            pl.BlockSpec((1, ROW_BLK, C), lambda i: (0, i, 0)),
            pl.BlockSpec((1, ROW_BLK, C), lambda i: (1, i, 0)),
            pl.BlockSpec((C, K * C), lambda i: (0, 0)),
        ],
        out_specs=pl.BlockSpec((ROW_BLK, K * C), lambda i: (i, 0)),
        out_shape=jax.ShapeDtypeStruct((N, K * C), jnp.float32),
    )(partials, partials, wcat)


def _final(partials):
    return pl.pallas_call(
        _final_body,
        grid=(N // ROW_BLK,),
        in_specs=[
            pl.BlockSpec((1, ROW_BLK, C), lambda i: (0, i, 0)),
            pl.BlockSpec((1, ROW_BLK, C), lambda i: (1, i, 0)),
        ],
        out_specs=pl.BlockSpec((ROW_BLK, C), lambda i: (i, 0)),
        out_shape=jax.ShapeDtypeStruct((N, C), jnp.float32),
    )(partials, partials)


@functools.cache
def _build_sc_scatter():
    mesh = plsc.VectorSubcoreMesh(core_axis_name="c", subcore_axis_name="s")

    @functools.partial(
        pl.kernel,
        out_type=jax.ShapeDtypeStruct((NC, N_PAD, C), jnp.float32),
        mesh=mesh,
        scratch_types=[
            pltpu.VMEM_SHARED((N_PAD, C), jnp.float32),
            pltpu.VMEM((CH, CHUNK), jnp.int32),
            pltpu.VMEM((CH, CHUNK), jnp.int32),
            pltpu.VMEM((2, GROUP_ROWS, C), jnp.float32),
            pltpu.SemaphoreType.DMA,
            pltpu.SemaphoreType.DMA,
            pltpu.SemaphoreType.DMA,
            pltpu.SemaphoreType.DMA,
        ],
        compiler_params=pltpu.CompilerParams(use_tc_tiling_on_sc=False),
    )
    def sc_scatter(y_hbm, gidx_hbm, dst_hbm, zeros_hbm, out_hbm,
                   acc, gi_v, di_v, rows_v, gs0, gs1, ss0, ss1):
        c = lax.axis_index("c")
        s = lax.axis_index("s")
        wid = c * NS + s
        r0 = s * RPT
        pltpu.sync_copy(zeros_hbm.at[pl.ds(r0, RPT)], acc.at[pl.ds(r0, RPT)])
        pltpu.sync_copy(gidx_hbm.at[wid], gi_v)
        pltpu.sync_copy(dst_hbm.at[wid], di_v)
        plsc.subcore_barrier()

        def fire_g(g, bank, sem):
            for b in range(GP):
                pltpu.async_copy(y_hbm.at[gi_v.at[g * GP + b]],
                                 rows_v.at[bank, pl.ds(b * CHUNK, CHUNK)], sem)

        def fire_s(g, bank, sem):
            for b in range(GP):
                pltpu.async_copy(rows_v.at[bank, pl.ds(b * CHUNK, CHUNK)],
                                 acc.at[di_v.at[g * GP + b]], sem, add=True)

        def drain_g(sem):
            pltpu.make_async_copy(y_hbm.at[pl.ds(0, GROUP_ROWS)],
                                  rows_v.at[0], sem).wait()

        def drain_s(sem):
            pltpu.make_async_copy(rows_v.at[0],
                                  acc.at[pl.ds(0, GROUP_ROWS)], sem).wait()

        fire_g(0, 0, gs0)

        def body(i, carry):
            g0 = 2 * i
            drain_g(gs0)

            @pl.when(i > 0)
            def _():
                drain_s(ss1)

            fire_g(g0 + 1, 1, gs1)
            fire_s(g0, 0, ss0)
            drain_g(gs1)
            drain_s(ss0)

            @pl.when(i < NG // 2 - 1)
            def _():
                fire_g(g0 + 2, 0, gs0)

            fire_s(g0 + 1, 1, ss1)
            return carry

        lax.fori_loop(0, NG // 2, body, 0)
        drain_s(ss1)
        plsc.subcore_barrier()
        pltpu.sync_copy(acc.at[pl.ds(r0, RPT)], out_hbm.at[c, pl.ds(r0, RPT)])

    return sc_scatter


def _sc_scatter(y, gidx, dst3, zeros):
    return _build_sc_scatter()(y, gidx, dst3, zeros)


def kernel(x, edge_index, kern_id, W1, W2):
    src = edge_index[0]
    dst = edge_index[1]
    src_pad = jnp.pad(src, (0, E_PAD - E))
    kid_pad = jnp.pad(kern_id, (0, E_PAD - E))
    dst_pad = jnp.pad(dst, (0, E_PAD - E), constant_values=DST_PAD)

    dst3 = dst_pad.reshape(NW, CH, CHUNK)
    zeros = jnp.zeros((N_PAD, C), jnp.float32)

    w1cat = W1.transpose(1, 0, 2).reshape(C, K * C)
    w2cat = W2.transpose(1, 0, 2).reshape(C, K * C)

    y1, gidx2d = _mm1(x, w1cat, src_pad, kid_pad)
    gidx = gidx2d.reshape(NW, CH, CHUNK)
    y1 = y1.reshape(N * K, C)
    p1 = _sc_scatter(y1, gidx, dst3, zeros)
    y2 = _mm_mid(p1, w2cat).reshape(N * K, C)
    p2 = _sc_scatter(y2, gidx, dst3, zeros)
    return _final(p2)


# speedup vs baseline: 1.2270x; 1.2270x over previous
import functools

import jax
import jax.numpy as jnp
from jax import lax
from jax.experimental import pallas as pl
from jax.experimental.pallas import tpu as pltpu
from jax.experimental.pallas import tpu_sc as plsc

N = 50000
E = 800000
C = 16
K = 27

NC = 2
NS = 16
NW = NC * NS
CHUNK = 128
GP = 4
GROUP_ROWS = GP * CHUNK
CH = 200
NG = CH // GP
E_PAD = NW * CH * CHUNK
N_PAD = N + 48
RPT = N_PAD // NS
ROW_BLK = 2000
DST_PAD = N + 1


def _bf16_bits(y):
    u = jax.lax.bitcast_convert_type(y, jnp.int32)
    rnd = jax.lax.bitwise_and(jax.lax.shift_right_logical(u, 16), 1)
    return jax.lax.shift_right_logical(u + 0x7FFF + rnd, 16)


def _pack_pair(ye, yo):
    return jax.lax.bitwise_or(_bf16_bits(ye),
                              jax.lax.shift_left(_bf16_bits(yo), 16))


def _mm1_body(x_ref, we_ref, wo_ref, src_ref, kid_ref, y_ref, g_ref):
    ye = jnp.dot(x_ref[...], we_ref[...], preferred_element_type=jnp.float32)
    yo = jnp.dot(x_ref[...], wo_ref[...], preferred_element_type=jnp.float32)
    y_ref[...] = _pack_pair(ye, yo)
    g_ref[...] = src_ref[...] * K + kid_ref[...]


def _mm_mid_body(p0_ref, p1_ref, we_ref, wo_ref, y_ref):
    h = jnp.maximum(p0_ref[0] + p1_ref[0], 0.0)
    ye = jnp.dot(h, we_ref[...], preferred_element_type=jnp.float32)
    yo = jnp.dot(h, wo_ref[...], preferred_element_type=jnp.float32)
    y_ref[...] = _pack_pair(ye, yo)


def _final_body(p0_ref, p1_ref, o_ref):
    o_ref[...] = jnp.maximum(p0_ref[0] + p1_ref[0], 0.0)


def _mm1(x, we, wo, src_pad, kid_pad):
    grid_n = N // ROW_BLK
    irows = E_PAD // 128
    iblk = irows // grid_n
    return pl.pallas_call(
        _mm1_body,
        grid=(grid_n,),
        in_specs=[
            pl.BlockSpec((ROW_BLK, C), lambda i: (i, 0)),
            pl.BlockSpec((C, K * C // 2), lambda i: (0, 0)),
            pl.BlockSpec((C, K * C // 2), lambda i: (0, 0)),
            pl.BlockSpec((iblk, 128), lambda i: (i, 0)),
            pl.BlockSpec((iblk, 128), lambda i: (i, 0)),
        ],
        out_specs=[
            pl.BlockSpec((ROW_BLK, K * C // 2), lambda i: (i, 0)),
            pl.BlockSpec((iblk, 128), lambda i: (i, 0)),
        ],
        out_shape=[
            jax.ShapeDtypeStruct((N, K * C // 2), jnp.int32),
            jax.ShapeDtypeStruct((irows, 128), jnp.int32),
        ],
    )(x, we, wo, src_pad.reshape(irows, 128), kid_pad.reshape(irows, 128))


def _mm_mid(partials, we, wo):
    return pl.pallas_call(
        _mm_mid_body,
        grid=(N // ROW_BLK,),
        in_specs=[
            pl.BlockSpec((1, ROW_BLK, C), lambda i: (0, i, 0)),
            pl.BlockSpec((1, ROW_BLK, C), lambda i: (1, i, 0)),
            pl.BlockSpec((C, K * C // 2), lambda i: (0, 0)),
            pl.BlockSpec((C, K * C // 2), lambda i: (0, 0)),
        ],
        out_specs=pl.BlockSpec((ROW_BLK, K * C // 2), lambda i: (i, 0)),
        out_shape=jax.ShapeDtypeStruct((N, K * C // 2), jnp.int32),
    )(partials, partials, we, wo)


def _final(partials):
    return pl.pallas_call(
        _final_body,
        grid=(N // ROW_BLK,),
        in_specs=[
            pl.BlockSpec((1, ROW_BLK, C), lambda i: (0, i, 0)),
            pl.BlockSpec((1, ROW_BLK, C), lambda i: (1, i, 0)),
        ],
        out_specs=pl.BlockSpec((ROW_BLK, C), lambda i: (i, 0)),
        out_shape=jax.ShapeDtypeStruct((N, C), jnp.float32),
    )(partials, partials)


@functools.cache
def _build_sc_scatter():
    mesh = plsc.VectorSubcoreMesh(core_axis_name="c", subcore_axis_name="s")

    @functools.partial(
        pl.kernel,
        out_type=jax.ShapeDtypeStruct((NC, N_PAD, C), jnp.float32),
        mesh=mesh,
        scratch_types=[
            pltpu.VMEM_SHARED((N_PAD, C), jnp.float32),
            pltpu.VMEM((CH, CHUNK), jnp.int32),
            pltpu.VMEM((CH, CHUNK), jnp.int32),
            pltpu.VMEM((2, GROUP_ROWS, C // 2), jnp.int32),
            pltpu.VMEM((2, GROUP_ROWS, C), jnp.float32),
            pltpu.SemaphoreType.DMA,
            pltpu.SemaphoreType.DMA,
            pltpu.SemaphoreType.DMA,
            pltpu.SemaphoreType.DMA,
        ],
        compiler_params=pltpu.CompilerParams(use_tc_tiling_on_sc=False,
                                             needs_layout_passes=False),
    )
    def sc_scatter(y_hbm, gidx_hbm, dst_hbm, zeros_hbm, out_hbm,
                   acc, gi_v, di_v, pk_v, rows_v, gs0, gs1, ss0, ss1):
        c = lax.axis_index("c")
        s = lax.axis_index("s")
        wid = c * NS + s
        r0 = s * RPT
        pltpu.sync_copy(zeros_hbm.at[pl.ds(r0, RPT)], acc.at[pl.ds(r0, RPT)])
        pltpu.sync_copy(gidx_hbm.at[wid], gi_v)
        pltpu.sync_copy(dst_hbm.at[wid], di_v)
        plsc.subcore_barrier()

        def fire_g(g, bank, sem):
            for b in range(GP):
                pltpu.async_copy(y_hbm.at[gi_v.at[g * GP + b]],
                                 pk_v.at[bank, pl.ds(b * CHUNK, CHUNK)], sem)

        def fire_s(g, bank, sem):
            for b in range(GP):
                pltpu.async_copy(rows_v.at[bank, pl.ds(b * CHUNK, CHUNK)],
                                 acc.at[di_v.at[g * GP + b]], sem, add=True)

        def drain_g(sem):
            pltpu.make_async_copy(y_hbm.at[pl.ds(0, GROUP_ROWS)],
                                  pk_v.at[0], sem).wait()

        def drain_s(sem):
            pltpu.make_async_copy(rows_v.at[0],
                                  acc.at[pl.ds(0, GROUP_ROWS)], sem).wait()

        ii = lax.iota(jnp.int32, 16)
        half = lax.shift_right_logical(ii, 3)
        col8 = lax.bitwise_and(ii, 7)
        cole = col8 * 2
        colo = cole + 1
        himask = jnp.int32(-65536)

        def convert(bank):
            pk_b = pk_v.at[bank]
            ro_b = rows_v.at[bank]

            def cbody(r, carry):
                for u in range(4):
                    rowsel = (r * 4 + u) * 2 + half
                    w = plsc.load_gather(pk_b, [rowsel, col8])
                    ev = plsc.bitcast(lax.shift_left(w, 16), jnp.float32)
                    od = plsc.bitcast(lax.bitwise_and(w, himask), jnp.float32)
                    plsc.store_scatter(ro_b, [rowsel, cole], ev)
                    plsc.store_scatter(ro_b, [rowsel, colo], od)
                return carry

            lax.fori_loop(0, GROUP_ROWS // 8, cbody, 0)

        fire_g(0, 0, gs0)

        def body(i, carry):
            g0 = 2 * i

            @pl.when(i > 0)
            def _():
                drain_s(ss1)

            fire_g(g0 + 1, 1, gs1)
            drain_g(gs0)
            convert(0)
            fire_s(g0, 0, ss0)
            drain_g(gs1)
            convert(1)
            drain_s(ss0)

            @pl.when(i < NG // 2 - 1)
            def _():
                fire_g(g0 + 2, 0, gs0)

            fire_s(g0 + 1, 1, ss1)
            return carry

        lax.fori_loop(0, NG // 2, body, 0)
        drain_s(ss1)
        plsc.subcore_barrier()
        pltpu.sync_copy(acc.at[pl.ds(r0, RPT)], out_hbm.at[c, pl.ds(r0, RPT)])

    return sc_scatter


def _sc_scatter(y, gidx, dst3, zeros):
    return _build_sc_scatter()(y, gidx, dst3, zeros)


def kernel(x, edge_index, kern_id, W1, W2):
    src = edge_index[0]
    dst = edge_index[1]
    src_pad = jnp.pad(src, (0, E_PAD - E))
    kid_pad = jnp.pad(kern_id, (0, E_PAD - E))
    dst_pad = jnp.pad(dst, (0, E_PAD - E), constant_values=DST_PAD)

    dst3 = dst_pad.reshape(NW, CH, CHUNK)
    zeros = jnp.zeros((N_PAD, C), jnp.float32)

    w1cat = W1.transpose(1, 0, 2).reshape(C, K * C)
    w2cat = W2.transpose(1, 0, 2).reshape(C, K * C)
    w1e, w1o = w1cat[:, 0::2], w1cat[:, 1::2]
    w2e, w2o = w2cat[:, 0::2], w2cat[:, 1::2]

    y1, gidx2d = _mm1(x, w1e, w1o, src_pad, kid_pad)
    gidx = gidx2d.reshape(NW, CH, CHUNK)
    y1 = y1.reshape(N * K, C // 2)
    p1 = _sc_scatter(y1, gidx, dst3, zeros)
    y2 = _mm_mid(p1, w2e, w2o).reshape(N * K, C // 2)
    p2 = _sc_scatter(y2, gidx, dst3, zeros)
    return _final(p2)


# speedup vs baseline: 1.2286x; 1.0013x over previous
import functools

import jax
import jax.numpy as jnp
from jax import lax
from jax.experimental import pallas as pl
from jax.experimental.pallas import tpu as pltpu
from jax.experimental.pallas import tpu_sc as plsc

N = 50000
E = 800000
C = 16
K = 27

NC = 2
NS = 16
NW = NC * NS
CHUNK = 128
GP = 4
GROUP_ROWS = GP * CHUNK
CH = 200
NG = CH // GP
E_PAD = NW * CH * CHUNK
IDX_ROWS = E_PAD // 128
IDX_PAD = IDX_ROWS * 128
N_PAD = N + 48
RPT = N_PAD // NS
ROW_BLK = 2000
DST_PAD = N + 1


def _bf16_bits(y):
    u = jax.lax.bitcast_convert_type(y, jnp.int32)
    rnd = jax.lax.bitwise_and(jax.lax.shift_right_logical(u, 16), 1)
    return jax.lax.shift_right_logical(u + 0x7FFF + rnd, 16)


def _pack_pair(ye, yo):
    return jax.lax.bitwise_or(_bf16_bits(ye),
                              jax.lax.shift_left(_bf16_bits(yo), 16))


def _mm1_body(x_ref, we_ref, wo_ref, src_ref, kid_ref, y_ref, g_ref):
    ye = jnp.dot(x_ref[...], we_ref[...], preferred_element_type=jnp.float32)
    yo = jnp.dot(x_ref[...], wo_ref[...], preferred_element_type=jnp.float32)
    y_ref[...] = _pack_pair(ye, yo)
    g_ref[...] = src_ref[...] * K + kid_ref[...]


def _mm_mid_body(p0_ref, p1_ref, we_ref, wo_ref, y_ref):
    h = jnp.maximum(p0_ref[0] + p1_ref[0], 0.0)
    ye = jnp.dot(h, we_ref[...], preferred_element_type=jnp.float32)
    yo = jnp.dot(h, wo_ref[...], preferred_element_type=jnp.float32)
    y_ref[...] = _pack_pair(ye, yo)


def _final_body(p0_ref, p1_ref, o_ref):
    o_ref[...] = jnp.maximum(p0_ref[0] + p1_ref[0], 0.0)


def _mm1(x, we, wo, src_pad, kid_pad):
    grid_n = N // ROW_BLK
    irows = IDX_ROWS
    iblk = irows // grid_n
    return pl.pallas_call(
        _mm1_body,
        grid=(grid_n,),
        in_specs=[
            pl.BlockSpec((ROW_BLK, C), lambda i: (i, 0)),
            pl.BlockSpec((C, K * C // 2), lambda i: (0, 0)),
            pl.BlockSpec((C, K * C // 2), lambda i: (0, 0)),
            pl.BlockSpec((iblk, 128), lambda i: (i, 0)),
            pl.BlockSpec((iblk, 128), lambda i: (i, 0)),
        ],
        out_specs=[
            pl.BlockSpec((ROW_BLK, K * C // 2), lambda i: (i, 0)),
            pl.BlockSpec((iblk, 128), lambda i: (i, 0)),
        ],
        out_shape=[
            jax.ShapeDtypeStruct((N, K * C // 2), jnp.int32),
            jax.ShapeDtypeStruct((irows, 128), jnp.int32),
        ],
    )(x, we, wo, src_pad.reshape(irows, 128), kid_pad.reshape(irows, 128))


def _mm_mid(partials, we, wo):
    return pl.pallas_call(
        _mm_mid_body,
        grid=(N // ROW_BLK,),
        in_specs=[
            pl.BlockSpec((1, ROW_BLK, C), lambda i: (0, i, 0)),
            pl.BlockSpec((1, ROW_BLK, C), lambda i: (1, i, 0)),
            pl.BlockSpec((C, K * C // 2), lambda i: (0, 0)),
            pl.BlockSpec((C, K * C // 2), lambda i: (0, 0)),
        ],
        out_specs=pl.BlockSpec((ROW_BLK, K * C // 2), lambda i: (i, 0)),
        out_shape=jax.ShapeDtypeStruct((N, K * C // 2), jnp.int32),
    )(partials, partials, we, wo)


def _final(partials):
    return pl.pallas_call(
        _final_body,
        grid=(N // ROW_BLK,),
        in_specs=[
            pl.BlockSpec((1, ROW_BLK, C), lambda i: (0, i, 0)),
            pl.BlockSpec((1, ROW_BLK, C), lambda i: (1, i, 0)),
        ],
        out_specs=pl.BlockSpec((ROW_BLK, C), lambda i: (i, 0)),
        out_shape=jax.ShapeDtypeStruct((N, C), jnp.float32),
    )(partials, partials)


@functools.cache
def _build_sc_scatter():
    mesh = plsc.VectorSubcoreMesh(core_axis_name="c", subcore_axis_name="s")

    @functools.partial(
        pl.kernel,
        out_type=jax.ShapeDtypeStruct((NC, N_PAD, C), jnp.float32),
        mesh=mesh,
        scratch_types=[
            pltpu.VMEM_SHARED((N_PAD, C), jnp.float32),
            pltpu.VMEM((CH, CHUNK), jnp.int32),
            pltpu.VMEM((CH, CHUNK), jnp.int32),
            pltpu.VMEM((2, GROUP_ROWS, C // 2), jnp.int32),
            pltpu.VMEM((2, GROUP_ROWS, C), jnp.float32),
            pltpu.SemaphoreType.DMA,
            pltpu.SemaphoreType.DMA,
            pltpu.SemaphoreType.DMA,
            pltpu.SemaphoreType.DMA,
        ],
        compiler_params=pltpu.CompilerParams(use_tc_tiling_on_sc=False,
                                             needs_layout_passes=False),
    )
    def sc_scatter(y_hbm, gidx_hbm, dst_hbm, zeros_hbm, out_hbm,
                   acc, gi_v, di_v, pk_v, rows_v, gs0, gs1, ss0, ss1):
        c = lax.axis_index("c")
        s = lax.axis_index("s")
        wid = c * NS + s
        r0 = s * RPT
        pltpu.sync_copy(zeros_hbm.at[pl.ds(r0, RPT)], acc.at[pl.ds(r0, RPT)])
        pltpu.sync_copy(gidx_hbm.at[wid], gi_v)
        pltpu.sync_copy(dst_hbm.at[wid], di_v)
        plsc.subcore_barrier()

        def fire_g(g, bank, sem):
            for b in range(GP):
                pltpu.async_copy(y_hbm.at[gi_v.at[g * GP + b]],
                                 pk_v.at[bank, pl.ds(b * CHUNK, CHUNK)], sem)

        def fire_s(g, bank, sem):
            for b in range(GP):
                pltpu.async_copy(rows_v.at[bank, pl.ds(b * CHUNK, CHUNK)],
                                 acc.at[di_v.at[g * GP + b]], sem, add=True)

        def drain_g(sem):
            pltpu.make_async_copy(y_hbm.at[pl.ds(0, GROUP_ROWS)],
                                  pk_v.at[0], sem).wait()

        def drain_s(sem):
            pltpu.make_async_copy(rows_v.at[0],
                                  acc.at[pl.ds(0, GROUP_ROWS)], sem).wait()

        ii = lax.iota(jnp.int32, 16)
        half = lax.shift_right_logical(ii, 3)
        col8 = lax.bitwise_and(ii, 7)
        cole = col8 * 2
        colo = cole + 1
        himask = jnp.int32(-65536)

        def convert(bank):
            pk_b = pk_v.at[bank]
            ro_b = rows_v.at[bank]

            def cbody(r, carry):
                for u in range(4):
                    rowsel = (r * 4 + u) * 2 + half
                    w = plsc.load_gather(pk_b, [rowsel, col8])
                    ev = plsc.bitcast(lax.shift_left(w, 16), jnp.float32)
                    od = plsc.bitcast(lax.bitwise_and(w, himask), jnp.float32)
                    plsc.store_scatter(ro_b, [rowsel, cole], ev)
                    plsc.store_scatter(ro_b, [rowsel, colo], od)
                return carry

            lax.fori_loop(0, GROUP_ROWS // 8, cbody, 0)

        fire_g(0, 0, gs0)

        def body(i, carry):
            g0 = 2 * i

            @pl.when(i > 0)
            def _():
                drain_s(ss1)

            fire_g(g0 + 1, 1, gs1)
            drain_g(gs0)
            convert(0)
            fire_s(g0, 0, ss0)
            drain_g(gs1)
            convert(1)
            drain_s(ss0)

            @pl.when(i < NG // 2 - 1)
            def _():
                fire_g(g0 + 2, 0, gs0)

            fire_s(g0 + 1, 1, ss1)
            return carry

        lax.fori_loop(0, NG // 2, body, 0)
        drain_s(ss1)
        plsc.subcore_barrier()
        pltpu.sync_copy(acc.at[pl.ds(r0, RPT)], out_hbm.at[c, pl.ds(r0, RPT)])

    return sc_scatter


def _sc_scatter(y, gidx, dst3, zeros):
    return _build_sc_scatter()(y, gidx, dst3, zeros)


def kernel(x, edge_index, kern_id, W1, W2):
    src = edge_index[0]
    dst = edge_index[1]
    src_pad = jnp.pad(src, (0, IDX_PAD - E))
    kid_pad = jnp.pad(kern_id, (0, IDX_PAD - E))
    dst_pad = jnp.pad(dst, (0, E_PAD - E), constant_values=DST_PAD)

    dst3 = dst_pad.reshape(NW, CH, CHUNK)
    zeros = jnp.zeros((N_PAD, C), jnp.float32)

    w1cat = W1.transpose(1, 0, 2).reshape(C, K * C)
    w2cat = W2.transpose(1, 0, 2).reshape(C, K * C)
    w1e, w1o = w1cat[:, 0::2], w1cat[:, 1::2]
    w2e, w2o = w2cat[:, 0::2], w2cat[:, 1::2]

    y1, gidx2d = _mm1(x, w1e, w1o, src_pad, kid_pad)
    gidx = gidx2d.reshape(-1)[:E_PAD].reshape(NW, CH, CHUNK)
    y1 = y1.reshape(N * K, C // 2)
    p1 = _sc_scatter(y1, gidx, dst3, zeros)
    y2 = _mm_mid(p1, w2e, w2o).reshape(N * K, C // 2)
    p2 = _sc_scatter(y2, gidx, dst3, zeros)
    return _final(p2)
